# baseline (speedup 1.0000x reference)
import jax
import jax.numpy as jnp
from jax import lax
from jax.experimental import pallas as pl
from jax.experimental.pallas import tpu as pltpu
from jax.experimental.pallas import tpu_sc as plsc

N = 10000
E = 320000
IN = 128
OUT = 128
R = 8

C = 128
NROWS = E // C
NW = 32
BASE_NCH = NROWS // NW
EXTRA = NROWS - BASE_NCH * NW
N_ACC = 10240
ROWS_PER_TILE = N_ACC // 16
MM_BLK = 2000


def _mm_body(x_ref, w_ref, o_ref):
    o_ref[0] = jnp.dot(x_ref[...], w_ref[0], preferred_element_type=jnp.float32)


def _transform(x, weight):
    return pl.pallas_call(
        _mm_body,
        grid=(N // MM_BLK, R),
        in_specs=[
            pl.BlockSpec((MM_BLK, IN), lambda i, r: (i, 0)),
            pl.BlockSpec((1, IN, OUT), lambda i, r: (r, 0, 0)),
        ],
        out_specs=pl.BlockSpec((1, MM_BLK, OUT), lambda i, r: (r, i, 0)),
        out_shape=jax.ShapeDtypeStruct((R, N, OUT), jnp.float32),
    )(x, weight)


def _add_body(a_ref, b_ref, o_ref):
    o_ref[...] = a_ref[0] + b_ref[0]


def _combine(partials):
    blk = 2000
    return pl.pallas_call(
        _add_body,
        grid=(N // blk,),
        in_specs=[
            pl.BlockSpec((1, blk, OUT), lambda i: (0, i, 0)),
            pl.BlockSpec((1, blk, OUT), lambda i: (1, i, 0)),
        ],
        out_specs=pl.BlockSpec((blk, OUT), lambda i: (i, 0)),
        out_shape=jax.ShapeDtypeStruct((N, OUT), jnp.float32),
    )(partials, partials)


def _sc_body(t_hbm, src_hbm, rel_hbm, dst_hbm, norm_hbm, zeros_hbm, out_hbm,
             gidx2, src2, rel2, dst2, norm2, rows_a, rows_b, accum,
             sem_ga, sem_gb, sem_sra, sem_srb, sem_dna, sem_dnb):
    cid = lax.axis_index("c")
    sid = lax.axis_index("s")
    wid = cid * 16 + sid
    nch = BASE_NCH + (wid < EXTRA).astype(jnp.int32)
    ew = (BASE_NCH * wid + jnp.minimum(wid, EXTRA)) * C

    def _issue_sr(t, slot, sem):
        off = ew + t * C
        pltpu.async_copy(src_hbm.at[pl.ds(off, C)], src2.at[slot], sem)
        pltpu.async_copy(rel_hbm.at[pl.ds(off, C)], rel2.at[slot], sem)

    def _issue_dn(t, slot, sem):
        off = ew + t * C
        pltpu.async_copy(dst_hbm.at[pl.ds(off, C)], dst2.at[slot], sem)
        pltpu.async_copy(norm_hbm.at[pl.ds(off, C)], norm2.at[slot], sem)

    def _wait_sr(slot, sem):
        pltpu.make_async_copy(src_hbm.at[pl.ds(0, C)], src2.at[slot], sem).wait()
        pltpu.make_async_copy(rel_hbm.at[pl.ds(0, C)], rel2.at[slot], sem).wait()

    def _wait_dn(slot, sem):
        pltpu.make_async_copy(dst_hbm.at[pl.ds(0, C)], dst2.at[slot], sem).wait()
        pltpu.make_async_copy(norm_hbm.at[pl.ds(0, C)], norm2.at[slot], sem).wait()

    def _gidx_and_gather(slot, rows, gsem):
        for j in range(C // 16):
            sl = pl.ds(j * 16, 16)
            gidx2[slot, sl] = rel2[slot, sl] * N + src2[slot, sl]
        pltpu.async_copy(t_hbm.at[gidx2.at[slot]], rows, gsem)

    _issue_sr(0, 0, sem_sra)
    _issue_dn(0, 0, sem_dna)
    _issue_sr(1, 1, sem_srb)
    _issue_dn(1, 1, sem_dnb)
    _wait_sr(0, sem_sra)
    _gidx_and_gather(0, rows_a, sem_ga)
    pltpu.sync_copy(zeros_hbm, accum.at[pl.ds(sid * ROWS_PER_TILE, ROWS_PER_TILE)])
    with jax.named_scope("sc_zero_barrier"):
        plsc.subcore_barrier()

    def _half(t, rows, rows_nxt, gsem, gsem_nxt, p):
        po = 1 - p
        pltpu.make_async_copy(t_hbm.at[gidx2.at[p]], rows, gsem).wait()
        _wait_dn(p, sem_dna if p == 0 else sem_dnb)
        @pl.loop(0, C // 16)
        def _grp(g):
            nv = norm2[p, pl.ds(g * 16, 16)]
            for i in range(16):
                nb = nv[i]
                for c8 in range(OUT // 16):
                    csl = pl.ds(c8 * 16, 16)
                    rows[g * 16 + i, csl] = rows[g * 16 + i, csl] * nb
        pltpu.sync_copy(rows, accum.at[dst2.at[p]], add=True)
        @pl.when(t + 1 < nch)
        def _():
            _wait_sr(po, sem_sra if po == 0 else sem_srb)
            _gidx_and_gather(po, rows_nxt, gsem_nxt)
        @pl.when(t + 2 < nch)
        def _():
            _issue_sr(t + 2, p, sem_sra if p == 0 else sem_srb)
            _issue_dn(t + 2, p, sem_dna if p == 0 else sem_dnb)

    with jax.named_scope("sc_edge_loop"):
        @pl.loop(0, (nch + 1) // 2)
        def _pair(k):
            _half(2 * k, rows_a, rows_b, sem_ga, sem_gb, 0)
            @pl.when(2 * k + 1 < nch)
            def _():
                _half(2 * k + 1, rows_b, rows_a, sem_gb, sem_ga, 1)

    with jax.named_scope("sc_dump"):
        plsc.subcore_barrier()
        orows = pl.ds(sid * ROWS_PER_TILE, ROWS_PER_TILE)
        pltpu.sync_copy(accum.at[orows], out_hbm.at[cid, orows])


def _sc_edge_pass(t_flat, src_e, rel_e, dst_e, norm_e, zeros):
    mesh = plsc.VectorSubcoreMesh(core_axis_name="c", subcore_axis_name="s")
    return pl.kernel(
        _sc_body,
        out_type=jax.ShapeDtypeStruct((2, N_ACC, OUT), jnp.float32),
        mesh=mesh,
        scratch_types=[
            pltpu.VMEM((2, C), jnp.int32),
            pltpu.VMEM((2, C), jnp.int32),
            pltpu.VMEM((2, C), jnp.int32),
            pltpu.VMEM((2, C), jnp.int32),
            pltpu.VMEM((2, C), jnp.float32),
            pltpu.VMEM((C, OUT), jnp.float32),
            pltpu.VMEM((C, OUT), jnp.float32),
            pltpu.VMEM_SHARED((N_ACC, OUT), jnp.float32),
            pltpu.SemaphoreType.DMA,
            pltpu.SemaphoreType.DMA,
            pltpu.SemaphoreType.DMA,
            pltpu.SemaphoreType.DMA,
            pltpu.SemaphoreType.DMA,
            pltpu.SemaphoreType.DMA,
        ],
    )(t_flat, src_e, rel_e, dst_e, norm_e, zeros)


def kernel(x, weight, norm, edge_index, rel_type):
    src = edge_index[0]
    dst = edge_index[1]
    norm_f = norm[:, 0]

    t = _transform(x, weight)
    t_flat = t.reshape(R * N, OUT)
    zeros = jnp.zeros((ROWS_PER_TILE, OUT), jnp.float32)
    partials = _sc_edge_pass(t_flat, src, rel_type, dst, norm_f, zeros)
    return _combine(partials)


# speedup vs baseline: 1.2623x; 1.2623x over previous
import jax
import jax.numpy as jnp
from jax import lax
from jax.experimental import pallas as pl
from jax.experimental.pallas import tpu as pltpu
from jax.experimental.pallas import tpu_sc as plsc

N = 10000
E = 320000
IN = 128
OUT = 128
R = 8

C = 128
NROWS = E // C
NW = 32
BASE_NCH = NROWS // NW
EXTRA = NROWS - BASE_NCH * NW
N_ACC = 10240
ROWS_PER_TILE = N_ACC // 16
MM_BLK = 2000


def _mm_body(x_ref, w_ref, o_ref):
    o_ref[0] = jnp.dot(x_ref[...], w_ref[0], preferred_element_type=jnp.float32)


def _transform(x, weight):
    return pl.pallas_call(
        _mm_body,
        grid=(N // MM_BLK, R),
        in_specs=[
            pl.BlockSpec((MM_BLK, IN), lambda i, r: (i, 0)),
            pl.BlockSpec((1, IN, OUT), lambda i, r: (r, 0, 0)),
        ],
        out_specs=pl.BlockSpec((1, MM_BLK, OUT), lambda i, r: (r, i, 0)),
        out_shape=jax.ShapeDtypeStruct((R, N, OUT), jnp.float32),
    )(x, weight)


def _add_body(a_ref, b_ref, o_ref):
    o_ref[...] = a_ref[0] + b_ref[0]


def _combine(partials):
    blk = 2000
    return pl.pallas_call(
        _add_body,
        grid=(N // blk,),
        in_specs=[
            pl.BlockSpec((1, blk, OUT), lambda i: (0, i, 0)),
            pl.BlockSpec((1, blk, OUT), lambda i: (1, i, 0)),
        ],
        out_specs=pl.BlockSpec((blk, OUT), lambda i: (i, 0)),
        out_shape=jax.ShapeDtypeStruct((N, OUT), jnp.float32),
    )(partials, partials)


def _sc_body(t_hbm, src_hbm, rel_hbm, dst_hbm, norm_hbm, zeros_hbm, out_hbm,
             gidx2, src2, rel2, dst2, norm2, rows_a, rows_b, accum,
             sem_ga, sem_gb, sem_sra, sem_srb, sem_dna, sem_dnb):
    cid = lax.axis_index("c")
    sid = lax.axis_index("s")
    wid = cid * 16 + sid
    nch = BASE_NCH + (wid < EXTRA).astype(jnp.int32)
    ew = (BASE_NCH * wid + jnp.minimum(wid, EXTRA)) * C

    def _issue_sr(t, slot, sem):
        off = ew + t * C
        pltpu.async_copy(src_hbm.at[pl.ds(off, C)], src2.at[slot], sem)
        pltpu.async_copy(rel_hbm.at[pl.ds(off, C)], rel2.at[slot], sem)

    def _issue_dn(t, slot, sem):
        off = ew + t * C
        pltpu.async_copy(dst_hbm.at[pl.ds(off, C)], dst2.at[slot], sem)
        pltpu.async_copy(norm_hbm.at[pl.ds(off, C)], norm2.at[slot], sem)

    def _wait_sr(slot, sem):
        pltpu.make_async_copy(src_hbm.at[pl.ds(0, C)], src2.at[slot], sem).wait()
        pltpu.make_async_copy(rel_hbm.at[pl.ds(0, C)], rel2.at[slot], sem).wait()

    def _wait_dn(slot, sem):
        pltpu.make_async_copy(dst_hbm.at[pl.ds(0, C)], dst2.at[slot], sem).wait()
        pltpu.make_async_copy(norm_hbm.at[pl.ds(0, C)], norm2.at[slot], sem).wait()

    def _gidx_and_gather(slot, rows, gsem):
        for j in range(C // 16):
            sl = pl.ds(j * 16, 16)
            gidx2[slot, sl] = rel2[slot, sl] * N + src2[slot, sl]
        pltpu.async_copy(t_hbm.at[gidx2.at[slot]], rows, gsem)

    _issue_sr(0, 0, sem_sra)
    _issue_dn(0, 0, sem_dna)
    _issue_sr(1, 1, sem_srb)
    _issue_dn(1, 1, sem_dnb)
    _wait_sr(0, sem_sra)
    _gidx_and_gather(0, rows_a, sem_ga)
    pltpu.sync_copy(zeros_hbm, accum.at[pl.ds(sid * ROWS_PER_TILE, ROWS_PER_TILE)])
    with jax.named_scope("sc_zero_barrier"):
        plsc.subcore_barrier()

    def _half(t, rows, rows_nxt, gsem, gsem_nxt, p):
        po = 1 - p
        pltpu.make_async_copy(t_hbm.at[gidx2.at[p]], rows, gsem).wait()
        @pl.when(t + 1 < nch)
        def _():
            _wait_sr(po, sem_sra if po == 0 else sem_srb)
            _gidx_and_gather(po, rows_nxt, gsem_nxt)
        _wait_dn(p, sem_dna if p == 0 else sem_dnb)
        @pl.loop(0, C // 16)
        def _grp(g):
            nv = norm2[p, pl.ds(g * 16, 16)]
            for i in range(16):
                nb = nv[i]
                for c8 in range(OUT // 16):
                    csl = pl.ds(c8 * 16, 16)
                    rows[g * 16 + i, csl] = rows[g * 16 + i, csl] * nb
        pltpu.sync_copy(rows, accum.at[dst2.at[p]], add=True)
        @pl.when(t + 2 < nch)
        def _():
            _issue_sr(t + 2, p, sem_sra if p == 0 else sem_srb)
            _issue_dn(t + 2, p, sem_dna if p == 0 else sem_dnb)

    with jax.named_scope("sc_edge_loop"):
        @pl.loop(0, (nch + 1) // 2)
        def _pair(k):
            _half(2 * k, rows_a, rows_b, sem_ga, sem_gb, 0)
            @pl.when(2 * k + 1 < nch)
            def _():
                _half(2 * k + 1, rows_b, rows_a, sem_gb, sem_ga, 1)

    with jax.named_scope("sc_dump"):
        plsc.subcore_barrier()
        orows = pl.ds(sid * ROWS_PER_TILE, ROWS_PER_TILE)
        pltpu.sync_copy(accum.at[orows], out_hbm.at[cid, orows])


def _sc_edge_pass(t_flat, src_e, rel_e, dst_e, norm_e, zeros):
    mesh = plsc.VectorSubcoreMesh(core_axis_name="c", subcore_axis_name="s")
    return pl.kernel(
        _sc_body,
        out_type=jax.ShapeDtypeStruct((2, N_ACC, OUT), jnp.float32),
        mesh=mesh,
        scratch_types=[
            pltpu.VMEM((2, C), jnp.int32),
            pltpu.VMEM((2, C), jnp.int32),
            pltpu.VMEM((2, C), jnp.int32),
            pltpu.VMEM((2, C), jnp.int32),
            pltpu.VMEM((2, C), jnp.float32),
            pltpu.VMEM((C, OUT), jnp.float32),
            pltpu.VMEM((C, OUT), jnp.float32),
            pltpu.VMEM_SHARED((N_ACC, OUT), jnp.float32),
            pltpu.SemaphoreType.DMA,
            pltpu.SemaphoreType.DMA,
            pltpu.SemaphoreType.DMA,
            pltpu.SemaphoreType.DMA,
            pltpu.SemaphoreType.DMA,
            pltpu.SemaphoreType.DMA,
        ],
    )(t_flat, src_e, rel_e, dst_e, norm_e, zeros)


def kernel(x, weight, norm, edge_index, rel_type):
    src = edge_index[0]
    dst = edge_index[1]
    norm_f = norm[:, 0]

    t = _transform(x, weight)
    t_flat = t.reshape(R * N, OUT)
    zeros = jnp.zeros((ROWS_PER_TILE, OUT), jnp.float32)
    partials = _sc_edge_pass(t_flat, src, rel_type, dst, norm_f, zeros)
    return _combine(partials)
